# baseline (speedup 1.0000x reference)
import dataclasses
import functools

import jax
import jax.numpy as jnp
from jax import lax
from jax.experimental import pallas as pl
from jax.experimental.pallas import tpu as pltpu
from jax.experimental.pallas import tpu_sc as plsc

_K = 8192
_D = 256
_N = 32 * 1024
_CC = 0.25

_RB = 1024
_KB = 1024

_NW = 32
_GC2 = 128
_NB = 3


def _argmin_body(x_ref, e_ref, idx_ref, minv_ref, rowsq_s, hesq_s, argf_s):
    r = pl.program_id(0)
    kb = pl.program_id(1)
    nk = pl.num_programs(1)
    x = x_ref[...]

    @pl.when(kb == 0)
    def _():
        rowsq_s[...] = jnp.sum(x * x, axis=1, keepdims=True)

    @pl.when(r == 0)
    def _():
        e = e_ref[...]
        hesq_s[0:1, pl.ds(kb * _KB, _KB)] = (0.5 * jnp.sum(e * e, axis=1))[None, :]
        hesq_s[1:2, pl.ds(kb * _KB, _KB)] = (
            lax.broadcasted_iota(jnp.int32, (1, _KB), 1).astype(jnp.float32)
            + (kb * _KB))

    xe = lax.dot_general(x, e_ref[...], (((1,), (1,)), ((), ())),
                         preferred_element_type=jnp.float32)
    c = xe - hesq_s[0:1, pl.ds(kb * _KB, _KB)]
    lmax = jnp.max(c, axis=1, keepdims=True)
    ids = hesq_s[1:2, pl.ds(kb * _KB, _KB)]
    larg = jnp.min(jnp.where(c == lmax, ids, float(_K)), axis=1,
                   keepdims=True)

    @pl.when(kb == 0)
    def _():
        minv_ref[...] = lmax
        argf_s[...] = larg

    @pl.when(kb > 0)
    def _():
        upd = lmax > minv_ref[...]
        minv_ref[...] = jnp.where(upd, lmax, minv_ref[...])
        argf_s[...] = jnp.where(upd, larg, argf_s[...])

    @pl.when(kb == nk - 1)
    def _():
        minv_ref[...] = rowsq_s[...] - 2.0 * minv_ref[...]
        idx_ref[...] = argf_s[...].astype(jnp.int32)


@functools.lru_cache(maxsize=None)
def _get_sc_gather_hist(nrows):
    ch = nrows // _NW
    ng = ch // _GC2
    mesh = plsc.VectorSubcoreMesh(core_axis_name="c", subcore_axis_name="s")
    cp = pltpu.CompilerParams()
    if "needs_layout_passes" in pltpu.CompilerParams.__dataclass_fields__:
        cp = dataclasses.replace(cp, needs_layout_passes=False)

    @functools.partial(
        pl.kernel,
        compiler_params=cp,
        out_type=[jax.ShapeDtypeStruct((nrows, _D), jnp.float32),
                  jax.ShapeDtypeStruct((_NW, _K), jnp.int32)],
        mesh=mesh,
        scratch_types=[pltpu.VMEM((ch,), jnp.int32),
                       pltpu.VMEM((_NB, _GC2, _D), jnp.float32),
                       pltpu.VMEM((_K,), jnp.int32)]
                      + [pltpu.SemaphoreType.DMA] * (2 * _NB),
    )
    def _sc_gather_hist(idx_hbm, emb_hbm, q_hbm, hist_hbm,
                        idx_v, rows_v, hist_v, *sems):
        sg, sw = sems[:_NB], sems[_NB:]
        wid = lax.axis_index("s") * 2 + lax.axis_index("c")
        base = wid * ch
        pltpu.sync_copy(idx_hbm.at[pl.ds(base, ch)], idx_v)

        gh = [None] * ng
        wh = [None] * ng
        for j in range(min(_NB - 1, ng)):
            gh[j] = pltpu.async_copy(
                emb_hbm.at[idx_v.at[pl.ds(j * _GC2, _GC2)]],
                rows_v.at[j], sg[j])

        @pl.loop(0, _K, step=16)
        def _(j):
            hist_v[pl.ds(j, 16)] = jnp.zeros((16,), jnp.int32)

        @pl.loop(0, ch, step=16)
        def _(j):
            plsc.addupdate_scatter(hist_v, [idx_v[pl.ds(j, 16)]],
                                   jnp.ones((16,), jnp.int32))

        pltpu.sync_copy(hist_v, hist_hbm.at[wid])

        waited = set()
        for g in range(ng):
            j = g + _NB - 1
            if j < ng:
                if g >= 1:
                    wh[g - 1].wait()
                    waited.add(g - 1)
                gh[j] = pltpu.async_copy(
                    emb_hbm.at[idx_v.at[pl.ds(j * _GC2, _GC2)]],
                    rows_v.at[j % _NB], sg[j % _NB])
            gh[g].wait()
            wh[g] = pltpu.async_copy(
                rows_v.at[g % _NB],
                q_hbm.at[pl.ds(base + g * _GC2, _GC2)], sw[g % _NB])
        for g in range(ng):
            if g not in waited:
                wh[g].wait()

    return _sc_gather_hist


def _stats_body(m0_ref, m1_ref, h0_ref, h1_ref, loss_ref, perp_ref):
    tot = jnp.sum(m0_ref[...]) + jnp.sum(m1_ref[...])
    loss_ref[...] = jnp.reshape(_CC * (tot / float(_N * _D)), (1, 1))
    counts = (jnp.sum(h0_ref[...].astype(jnp.float32), axis=0)
              + jnp.sum(h1_ref[...].astype(jnp.float32), axis=0))
    probs = counts / (jnp.sum(counts) + 1e-10)
    ent = jnp.sum(probs * jnp.log(probs + 1e-10))
    perp_ref[...] = jnp.reshape(jnp.exp(-ent), (1, 1))


def kernel(inputs, embedding):
    flat = inputs.reshape(-1, _D)
    nh = _N // 2
    nrb = nh // _RB
    parts = []
    for part in range(2):
        off = part * nrb
        idx2, minv2 = pl.pallas_call(
            _argmin_body,
            grid=(nrb, _K // _KB),
            in_specs=[pl.BlockSpec((_RB, _D),
                                   lambda r, k, off=off: (r + off, 0)),
                      pl.BlockSpec((_KB, _D), lambda r, k: (k, 0))],
            out_specs=[pl.BlockSpec((_RB, 1), lambda r, k: (r, 0)),
                       pl.BlockSpec((_RB, 1), lambda r, k: (r, 0))],
            out_shape=[jax.ShapeDtypeStruct((nh, 1), jnp.int32),
                       jax.ShapeDtypeStruct((nh, 1), jnp.float32)],
            scratch_shapes=[pltpu.VMEM((_RB, 1), jnp.float32),
                            pltpu.VMEM((2, _K), jnp.float32),
                            pltpu.VMEM((_RB, 1), jnp.float32)],
        )(flat, embedding)
        q, hist = _get_sc_gather_hist(nh)(idx2.reshape(nh), embedding)
        parts.append((idx2, minv2, q, hist))

    (i0, m0, q0, h0), (i1, m1, q1, h1) = parts
    loss2, perp2 = pl.pallas_call(
        _stats_body,
        in_specs=[pl.BlockSpec((nh // 128, 128), lambda: (0, 0)),
                  pl.BlockSpec((nh // 128, 128), lambda: (0, 0)),
                  pl.BlockSpec((_NW, _K), lambda: (0, 0)),
                  pl.BlockSpec((_NW, _K), lambda: (0, 0))],
        out_specs=[pl.BlockSpec((1, 1), lambda: (0, 0)),
                   pl.BlockSpec((1, 1), lambda: (0, 0))],
        out_shape=[jax.ShapeDtypeStruct((1, 1), jnp.float32),
                   jax.ShapeDtypeStruct((1, 1), jnp.float32)],
    )(m0.reshape(nh // 128, 128), m1.reshape(nh // 128, 128), h0, h1)

    quantized_st = jnp.concatenate([q0, q1], axis=0).reshape(inputs.shape)
    indices = jnp.concatenate([i0.reshape(nh), i1.reshape(nh)]).reshape(
        inputs.shape[0], inputs.shape[1])
    return (quantized_st, loss2[0, 0], indices, perp2[0, 0])


# speedup vs baseline: 1.0342x; 1.0342x over previous
import dataclasses
import functools

import jax
import jax.numpy as jnp
from jax import lax
from jax.experimental import pallas as pl
from jax.experimental.pallas import tpu as pltpu
from jax.experimental.pallas import tpu_sc as plsc

_K = 8192
_D = 256
_N = 32 * 1024
_CC = 0.25

_RB = 1024
_KB = 1024

_NW = 32
_CHUNK = _N // _NW
_GC = 256


def _argmin_body(x_ref, e_ref, idx_ref, minv_ref, rowsq_s, hesq_s, argf_s,
                 cmax_s):
    r = pl.program_id(0)
    kb = pl.program_id(1)
    nk = pl.num_programs(1)
    x = x_ref[...]

    @pl.when(kb == 0)
    def _():
        rowsq_s[...] = jnp.sum(x * x, axis=1, keepdims=True)

    @pl.when(r == 0)
    def _():
        e = e_ref[...]
        hesq_s[0:1, pl.ds(kb * _KB, _KB)] = (0.5 * jnp.sum(e * e, axis=1))[None, :]
        hesq_s[1:2, pl.ds(kb * _KB, _KB)] = (
            lax.broadcasted_iota(jnp.int32, (1, _KB), 1).astype(jnp.float32)
            + (kb * _KB))

    xe = lax.dot_general(x, e_ref[...], (((1,), (1,)), ((), ())),
                         preferred_element_type=jnp.float32)
    c = xe - hesq_s[0:1, pl.ds(kb * _KB, _KB)]
    lmax = jnp.max(c, axis=1, keepdims=True)
    ids = hesq_s[1:2, pl.ds(kb * _KB, _KB)]
    larg = jnp.min(jnp.where(c == lmax, ids, float(_K)), axis=1,
                   keepdims=True)

    @pl.when(kb == 0)
    def _():
        cmax_s[...] = lmax
        argf_s[...] = larg

    @pl.when(kb > 0)
    def _():
        upd = lmax > cmax_s[...]
        cmax_s[...] = jnp.where(upd, lmax, cmax_s[...])
        argf_s[...] = jnp.where(upd, larg, argf_s[...])

    @pl.when(kb == nk - 1)
    def _():
        minv = rowsq_s[...] - 2.0 * cmax_s[...]
        minv_ref[...] = jnp.transpose(minv)[None]
        idx_ref[...] = jnp.transpose(
            argf_s[...].astype(jnp.int32))[None]


@functools.lru_cache(maxsize=None)
def _get_sc_gather_hist():
    mesh = plsc.VectorSubcoreMesh(core_axis_name="c", subcore_axis_name="s")
    cp = pltpu.CompilerParams()
    if "needs_layout_passes" in pltpu.CompilerParams.__dataclass_fields__:
        cp = dataclasses.replace(cp, needs_layout_passes=False)

    @functools.partial(
        pl.kernel,
        compiler_params=cp,
        out_type=[jax.ShapeDtypeStruct((_N, _D), jnp.float32),
                  jax.ShapeDtypeStruct((_NW, _K), jnp.int32)],
        mesh=mesh,
        scratch_types=[pltpu.VMEM((_CHUNK,), jnp.int32),
                       pltpu.VMEM((_GC, _D), jnp.float32),
                       pltpu.VMEM((_K,), jnp.int32),
                       pltpu.SemaphoreType.DMA],
    )
    def _sc_gather_hist(idx_hbm, emb_hbm, q_hbm, hist_hbm,
                        idx_v, rows_v, hist_v, sem):
        wid = lax.axis_index("s") * 2 + lax.axis_index("c")
        base = wid * _CHUNK
        pltpu.sync_copy(idx_hbm.at[pl.ds(base, _CHUNK)], idx_v)

        @pl.loop(0, _K, step=16)
        def _(j):
            hist_v[pl.ds(j, 16)] = jnp.zeros((16,), jnp.int32)

        for g in range(_CHUNK // _GC):
            pltpu.async_copy(emb_hbm.at[idx_v.at[pl.ds(g * _GC, _GC)]],
                             rows_v, sem).wait()
            pltpu.sync_copy(rows_v, q_hbm.at[pl.ds(base + g * _GC, _GC)])

        @pl.loop(0, _CHUNK, step=16)
        def _(j):
            v = idx_v[pl.ds(j, 16)]
            plsc.addupdate_scatter(hist_v, [v], jnp.ones((16,), jnp.int32))

        pltpu.sync_copy(hist_v, hist_hbm.at[wid])

    return _sc_gather_hist


def _stats_body(minv_ref, hist_ref, loss_ref, perp_ref):
    loss = _CC * (jnp.sum(minv_ref[...]) / float(_N * _D))
    loss_ref[...] = jnp.reshape(loss, (1, 1))
    counts = jnp.sum(hist_ref[...].astype(jnp.float32), axis=0)
    probs = counts / (jnp.sum(counts) + 1e-10)
    ent = jnp.sum(probs * jnp.log(probs + 1e-10))
    perp_ref[...] = jnp.reshape(jnp.exp(-ent), (1, 1))


def kernel(inputs, embedding):
    flat = inputs.reshape(-1, _D)
    idx2, minv2 = pl.pallas_call(
        _argmin_body,
        grid=(_N // _RB, _K // _KB),
        in_specs=[pl.BlockSpec((_RB, _D), lambda r, k: (r, 0)),
                  pl.BlockSpec((_KB, _D), lambda r, k: (k, 0))],
        out_specs=[pl.BlockSpec((1, 1, _RB), lambda r, k: (r, 0, 0)),
                   pl.BlockSpec((1, 1, _RB), lambda r, k: (r, 0, 0))],
        out_shape=[jax.ShapeDtypeStruct((_N // _RB, 1, _RB), jnp.int32),
                   jax.ShapeDtypeStruct((_N // _RB, 1, _RB), jnp.float32)],
        scratch_shapes=[pltpu.VMEM((_RB, 1), jnp.float32),
                        pltpu.VMEM((2, _K), jnp.float32),
                        pltpu.VMEM((_RB, 1), jnp.float32),
                        pltpu.VMEM((_RB, 1), jnp.float32)],
    )(flat, embedding)
    indices_flat = idx2.reshape(_N)

    quantized_flat, hist = _get_sc_gather_hist()(indices_flat, embedding)

    loss2, perp2 = pl.pallas_call(
        _stats_body,
        in_specs=[pl.BlockSpec((_N // _RB, 1, _RB), lambda: (0, 0, 0)),
                  pl.BlockSpec((_NW, _K), lambda: (0, 0))],
        out_specs=[pl.BlockSpec((1, 1), lambda: (0, 0)),
                   pl.BlockSpec((1, 1), lambda: (0, 0))],
        out_shape=[jax.ShapeDtypeStruct((1, 1), jnp.float32),
                   jax.ShapeDtypeStruct((1, 1), jnp.float32)],
    )(minv2, hist)

    quantized_st = quantized_flat.reshape(inputs.shape)
    indices = indices_flat.reshape(inputs.shape[0], inputs.shape[1])
    return (quantized_st, loss2[0, 0], indices, perp2[0, 0])


# speedup vs baseline: 1.1582x; 1.1199x over previous
import dataclasses
import functools

import jax
import jax.numpy as jnp
from jax import lax
from jax.experimental import pallas as pl
from jax.experimental.pallas import tpu as pltpu
from jax.experimental.pallas import tpu_sc as plsc

_K = 8192
_D = 256
_N = 32 * 1024
_CC = 0.25

_RB = 1024
_KB = 2048

_NW = 32
_CHUNK = _N // _NW
_GC = 256


def _argmin_body(x_ref, e_ref, idx_ref, minv_ref, rowsq_s, hesq_s, argf_s,
                 cmax_s):
    r = pl.program_id(0)
    kb = pl.program_id(1)
    nk = pl.num_programs(1)
    x = x_ref[...]

    @pl.when(kb == 0)
    def _():
        rowsq_s[...] = jnp.sum(x * x, axis=1, keepdims=True)

    @pl.when(r == 0)
    def _():
        e = e_ref[...]
        hesq_s[0:1, pl.ds(kb * _KB, _KB)] = (0.5 * jnp.sum(e * e, axis=1))[None, :]
        hesq_s[1:2, pl.ds(kb * _KB, _KB)] = (
            lax.broadcasted_iota(jnp.int32, (1, _KB), 1).astype(jnp.float32)
            + (kb * _KB))

    xe = lax.dot_general(x, e_ref[...], (((1,), (1,)), ((), ())),
                         preferred_element_type=jnp.float32)
    c = xe - hesq_s[0:1, pl.ds(kb * _KB, _KB)]
    lmax = jnp.max(c, axis=1, keepdims=True)
    ids = hesq_s[1:2, pl.ds(kb * _KB, _KB)]
    larg = jnp.min(jnp.where(c == lmax, ids, float(_K)), axis=1,
                   keepdims=True)

    @pl.when(kb == 0)
    def _():
        cmax_s[...] = lmax
        argf_s[...] = larg

    @pl.when(kb > 0)
    def _():
        upd = lmax > cmax_s[...]
        cmax_s[...] = jnp.where(upd, lmax, cmax_s[...])
        argf_s[...] = jnp.where(upd, larg, argf_s[...])

    @pl.when(kb == nk - 1)
    def _():
        minv = rowsq_s[...] - 2.0 * cmax_s[...]
        minv_ref[...] = jnp.transpose(minv)[None]
        idx_ref[...] = jnp.transpose(
            argf_s[...].astype(jnp.int32))[None]


@functools.lru_cache(maxsize=None)
def _get_sc_gather_hist():
    mesh = plsc.VectorSubcoreMesh(core_axis_name="c", subcore_axis_name="s")
    cp = pltpu.CompilerParams()
    if "needs_layout_passes" in pltpu.CompilerParams.__dataclass_fields__:
        cp = dataclasses.replace(cp, needs_layout_passes=False)

    @functools.partial(
        pl.kernel,
        compiler_params=cp,
        out_type=[jax.ShapeDtypeStruct((_N, _D), jnp.float32),
                  jax.ShapeDtypeStruct((_NW, _K), jnp.int32)],
        mesh=mesh,
        scratch_types=[pltpu.VMEM((_CHUNK,), jnp.int32),
                       pltpu.VMEM((_GC, _D), jnp.float32),
                       pltpu.VMEM((_K,), jnp.int32),
                       pltpu.SemaphoreType.DMA],
    )
    def _sc_gather_hist(idx_hbm, emb_hbm, q_hbm, hist_hbm,
                        idx_v, rows_v, hist_v, sem):
        wid = lax.axis_index("s") * 2 + lax.axis_index("c")
        base = wid * _CHUNK
        pltpu.sync_copy(idx_hbm.at[pl.ds(base, _CHUNK)], idx_v)

        @pl.loop(0, _K, step=16)
        def _(j):
            hist_v[pl.ds(j, 16)] = jnp.zeros((16,), jnp.int32)

        for g in range(_CHUNK // _GC):
            pltpu.async_copy(emb_hbm.at[idx_v.at[pl.ds(g * _GC, _GC)]],
                             rows_v, sem).wait()
            pltpu.sync_copy(rows_v, q_hbm.at[pl.ds(base + g * _GC, _GC)])

        @pl.loop(0, _CHUNK, step=16)
        def _(j):
            v = idx_v[pl.ds(j, 16)]
            plsc.addupdate_scatter(hist_v, [v], jnp.ones((16,), jnp.int32))

        pltpu.sync_copy(hist_v, hist_hbm.at[wid])

    return _sc_gather_hist


def _stats_body(minv_ref, hist_ref, loss_ref, perp_ref):
    loss = _CC * (jnp.sum(minv_ref[...]) / float(_N * _D))
    loss_ref[...] = jnp.reshape(loss, (1, 1))
    counts = jnp.sum(hist_ref[...].astype(jnp.float32), axis=0)
    probs = counts / (jnp.sum(counts) + 1e-10)
    ent = jnp.sum(probs * jnp.log(probs + 1e-10))
    perp_ref[...] = jnp.reshape(jnp.exp(-ent), (1, 1))


def kernel(inputs, embedding):
    flat = inputs.reshape(-1, _D)
    idx2, minv2 = pl.pallas_call(
        _argmin_body,
        grid=(_N // _RB, _K // _KB),
        in_specs=[pl.BlockSpec((_RB, _D), lambda r, k: (r, 0)),
                  pl.BlockSpec((_KB, _D), lambda r, k: (k, 0))],
        out_specs=[pl.BlockSpec((1, 1, _RB), lambda r, k: (r, 0, 0)),
                   pl.BlockSpec((1, 1, _RB), lambda r, k: (r, 0, 0))],
        out_shape=[jax.ShapeDtypeStruct((_N // _RB, 1, _RB), jnp.int32),
                   jax.ShapeDtypeStruct((_N // _RB, 1, _RB), jnp.float32)],
        scratch_shapes=[pltpu.VMEM((_RB, 1), jnp.float32),
                        pltpu.VMEM((2, _K), jnp.float32),
                        pltpu.VMEM((_RB, 1), jnp.float32),
                        pltpu.VMEM((_RB, 1), jnp.float32)],
    )(flat, embedding)
    indices_flat = idx2.reshape(_N)

    quantized_flat, hist = _get_sc_gather_hist()(indices_flat, embedding)

    loss2, perp2 = pl.pallas_call(
        _stats_body,
        in_specs=[pl.BlockSpec((_N // _RB, 1, _RB), lambda: (0, 0, 0)),
                  pl.BlockSpec((_NW, _K), lambda: (0, 0))],
        out_specs=[pl.BlockSpec((1, 1), lambda: (0, 0)),
                   pl.BlockSpec((1, 1), lambda: (0, 0))],
        out_shape=[jax.ShapeDtypeStruct((1, 1), jnp.float32),
                   jax.ShapeDtypeStruct((1, 1), jnp.float32)],
    )(minv2, hist)

    quantized_st = quantized_flat.reshape(inputs.shape)
    indices = indices_flat.reshape(inputs.shape[0], inputs.shape[1])
    return (quantized_st, loss2[0, 0], indices, perp2[0, 0])
